# baseline (speedup 1.0000x reference)
import functools

import jax
import jax.numpy as jnp
from jax import lax
from jax.experimental import pallas as pl
from jax.experimental.pallas import tpu as pltpu
from jax.experimental.pallas import tpu_sc as plsc

D_MODEL = 2048
TOT_EXPERT = 64
TOP_K = 8
ALPHA = 1.0

BLOCK_R = 2048
N_WORKERS = 32
CHUNK = 32768 // N_WORKERS
HALF = CHUNK // 2


def _logits_block(w_ref, x_ref, avg_ref, out_ref):
    w = w_ref[...]
    x = x_ref[...]
    logits = jax.lax.dot_general(
        x, w,
        dimension_numbers=(((1,), (1,)), ((), ())),
        preferred_element_type=jnp.float32,
    )
    out_ref[...] = logits - ALPHA * avg_ref[...]


def _tc_logits(inp, W, avg2):
    n = inp.shape[0]
    return pl.pallas_call(
        _logits_block,
        grid=(n // BLOCK_R,),
        in_specs=[
            pl.BlockSpec((TOT_EXPERT, D_MODEL), lambda i: (0, 0)),
            pl.BlockSpec((BLOCK_R, D_MODEL), lambda i: (i, 0)),
            pl.BlockSpec((1, TOT_EXPERT), lambda i: (0, 0)),
        ],
        out_specs=pl.BlockSpec((BLOCK_R, TOT_EXPERT), lambda i: (i, 0)),
        out_shape=jax.ShapeDtypeStruct((n, TOT_EXPERT), jnp.float32),
    )(W, inp, avg2)


def _merge_top8(ak, av, bk, bv, low8):
    ck = jnp.where(low8, ak, lax.rev(bk, (0,)))
    cv = jnp.where(low8, av, lax.rev(bv, (0,)))
    return plsc.sort_key_val(ck, cv, descending=True)


def _sc_route(logits_hbm, idx_hbm, scr_hbm, loc, oidx, oscr):
    wid = lax.axis_index("s") * 2 + lax.axis_index("c")

    lane = lax.broadcasted_iota(jnp.int32, (16,), 0)
    low8 = lane < 8

    for h in range(CHUNK // HALF):
        pltpu.sync_copy(logits_hbm.at[wid, pl.ds(h * HALF, HALF)], loc)

        def body(t, carry):
            sk = []
            sv = []
            for c in range(4):
                k = loc[t, pl.ds(c * 16, 16)]
                v = lane + (c * 16)
                s = plsc.sort_key_val(k, v, descending=True)
                sk.append(s[0])
                sv.append(s[1])
            mk0, mv0 = _merge_top8(sk[0], sv[0], sk[1], sv[1], low8)
            mk1, mv1 = _merge_top8(sk[2], sv[2], sk[3], sv[3], low8)
            tk, tv = _merge_top8(mk0, mv0, mk1, mv1, low8)
            e = jnp.exp(tk - plsc.cummax(tk))
            em = jnp.where(low8, e, 0.0)
            s = em / jnp.sum(em)
            o = (h * HALF + t) * TOP_K
            plsc.store_compressed(oidx.at[pl.ds(o, 16)], tv, mask=low8)
            plsc.store_compressed(oscr.at[pl.ds(o, 16)], s, mask=low8)
            return carry

        lax.fori_loop(0, HALF, body, 0)
    pltpu.sync_copy(oidx.at[pl.ds(0, CHUNK * TOP_K)],
                    idx_hbm.at[wid])
    pltpu.sync_copy(oscr.at[pl.ds(0, CHUNK * TOP_K)],
                    scr_hbm.at[wid])


_sc_kernel = functools.partial(
    pl.kernel,
    mesh=plsc.VectorSubcoreMesh(core_axis_name="c", subcore_axis_name="s"),
    compiler_params=pltpu.CompilerParams(needs_layout_passes=False),
    out_type=[
        jax.ShapeDtypeStruct((N_WORKERS, CHUNK * TOP_K), jnp.int32),
        jax.ShapeDtypeStruct((N_WORKERS, CHUNK * TOP_K), jnp.float32),
    ],
    scratch_types=[
        pltpu.VMEM((HALF, TOT_EXPERT), jnp.float32),
        pltpu.VMEM((CHUNK * TOP_K + 16,), jnp.int32),
        pltpu.VMEM((CHUNK * TOP_K + 16,), jnp.float32),
    ],
)(_sc_route)


@jax.jit
def kernel(inp, W, avg_logits):
    n = inp.shape[0]
    avg2 = avg_logits.reshape(1, TOT_EXPERT)
    logits = _tc_logits(inp, W, avg2)
    logits3 = logits.reshape(N_WORKERS, CHUNK, TOT_EXPERT)
    idx2, scr2 = _sc_kernel(logits3)
    return (idx2.reshape(n, TOP_K), scr2.reshape(n, TOP_K))


# speedup vs baseline: 1.8761x; 1.8761x over previous
import functools

import jax
import jax.numpy as jnp
from jax.experimental import pallas as pl
from jax.experimental.pallas import tpu as pltpu

D_MODEL = 2048
TOT_EXPERT = 64
TOP_K = 8
ALPHA = 1.0

BLOCK_R = 2048


def _router_block(w_ref, x_ref, avg_ref, idx_ref, score_ref):
    w = w_ref[...]
    x = x_ref[...]
    logits = jax.lax.dot_general(
        w, x,
        dimension_numbers=(((1,), (1,)), ((), ())),
        preferred_element_type=jnp.float32,
    )
    vals = logits - ALPHA * avg_ref[...]

    row = jax.lax.broadcasted_iota(jnp.int32, vals.shape, 0)
    top_vals = []
    top_idx = []
    for _ in range(TOP_K):
        m = jnp.max(vals, axis=0, keepdims=True)
        eq = vals == m
        i = jnp.min(jnp.where(eq, row, TOT_EXPERT), axis=0, keepdims=True)
        top_vals.append(m)
        top_idx.append(i)
        vals = jnp.where(row == i, -jnp.inf, vals)

    tv = jnp.concatenate(top_vals, axis=0)
    ti = jnp.concatenate(top_idx, axis=0)
    e = jnp.exp(tv - tv[0:1, :])
    s = e / jnp.sum(e, axis=0, keepdims=True)
    idx_ref[...] = ti.T
    score_ref[...] = s.T


@functools.partial(jax.jit, static_argnames=())
def kernel(inp, W, avg_logits):
    n = inp.shape[0]
    grid = (n // BLOCK_R,)
    avg2 = avg_logits.reshape(TOT_EXPERT, 1)
    out_idx, out_score = pl.pallas_call(
        _router_block,
        grid=grid,
        in_specs=[
            pl.BlockSpec((TOT_EXPERT, D_MODEL), lambda i: (0, 0)),
            pl.BlockSpec((BLOCK_R, D_MODEL), lambda i: (i, 0)),
            pl.BlockSpec((TOT_EXPERT, 1), lambda i: (0, 0)),
        ],
        out_specs=[
            pl.BlockSpec((BLOCK_R, TOP_K), lambda i: (i, 0)),
            pl.BlockSpec((BLOCK_R, TOP_K), lambda i: (i, 0)),
        ],
        out_shape=[
            jax.ShapeDtypeStruct((n, TOP_K), jnp.int32),
            jax.ShapeDtypeStruct((n, TOP_K), jnp.float32),
        ],
        compiler_params=pltpu.CompilerParams(
            dimension_semantics=("parallel",),
        ),
    )(W, inp, avg2)
    return (out_idx, out_score)
